# baseline (speedup 1.0000x reference)
import functools

import jax
import jax.numpy as jnp
from jax import lax
from jax.experimental import pallas as pl
from jax.experimental.pallas import tpu as pltpu
from jax.experimental.pallas import tpu_sc as plsc

_HIDDEN = 128
_LANES = 16
_NW = 32
_K = 128
_GC = 8
_GK = _GC * _K


def _emb_body(seq_hbm, segf_hbm, tok_hbm, possum_hbm, d_hbm, out_hbm,
              possum_v, d_v, idx_grp, segf_g0, segf_g1, e2_v, bufs,
              sem_t0, sem_t1, sem_o0, sem_o1,
              sem_si0, sem_si1, sem_ss0, sem_ss1):
    sem_t = [sem_t0, sem_t1]
    sem_o = [sem_o0, sem_o1]
    sem_si = [sem_si0, sem_si1]
    sem_ss = [sem_ss0, sem_ss1]
    segf_grp = [segf_g0, segf_g1]
    n = out_hbm.shape[0]
    rows_per_w = n // _NW
    nchunk = rows_per_w // _K
    ngroup = nchunk // _GC
    sent = possum_v.shape[0]
    wid = lax.axis_index("s") * 2 + lax.axis_index("c")
    wbase = wid * rows_per_w

    pltpu.sync_copy(possum_hbm, possum_v)
    pltpu.sync_copy(d_hbm, d_v)
    dvs = [d_v[pl.ds(j * _LANES, _LANES)] for j in range(_HIDDEN // _LANES)]
    iota16 = lax.iota(jnp.int32, _LANES)

    def fire_stage(g, slot):
        pltpu.async_copy(seq_hbm.at[pl.ds(wbase + g * _GK, _GK)],
                         idx_grp.at[slot], sem_si[slot])
        pltpu.async_copy(segf_hbm.at[pl.ds(wbase + g * _GK, _GK)],
                         segf_grp[slot], sem_ss[slot])

    def wait_stage(g, slot):
        pltpu.make_async_copy(seq_hbm.at[pl.ds(wbase + g * _GK, _GK)],
                              idx_grp.at[slot], sem_si[slot]).wait()
        pltpu.make_async_copy(segf_hbm.at[pl.ds(wbase + g * _GK, _GK)],
                              segf_grp[slot], sem_ss[slot]).wait()

    def prefill(b, u, slot):
        s_base = (u % (sent // _K)) * _K
        segbase = u * _K
        bv = bufs.at[b]
        sv = segf_grp[slot]

        def expand(rb, carry):
            r0 = rb * _LANES
            sb = sv[pl.ds(segbase + r0, _LANES)]
            for c in range(_LANES):
                plsc.store_scatter(
                    e2_v, [r0 + iota16, jnp.full((_LANES,), c, jnp.int32)],
                    sb)
            return carry

        lax.fori_loop(0, _K // _LANES, expand, 0)

        def row(r, carry):
            spl = e2_v[r, :]
            for j in range(_HIDDEN // _LANES):
                sl = pl.ds(j * _LANES, _LANES)
                bv[r, sl] = possum_v[s_base + r, sl] + spl * dvs[j]
            return carry

        lax.fori_loop(0, _K, row, 0, unroll=4)

    def tok_idx_ref(u, slot):
        return idx_grp.at[slot, pl.ds(u * _K, _K)]

    def step(cc, b, u, slot):
        @pl.when(cc >= 2)
        def _():
            pltpu.make_async_copy(
                bufs.at[b],
                out_hbm.at[pl.ds(wbase + (cc - 2) * _K, _K)],
                sem_o[b]).wait()

        prefill(b, u, slot)

        if u > 0:
            iref = tok_idx_ref(u - 1, slot)
        else:
            iref = tok_idx_ref(_GC - 1, 1 - slot)

        @pl.when(cc >= 1)
        def _():
            pltpu.make_async_copy(tok_hbm.at[iref], bufs.at[1 - b],
                                  sem_t[1 - b]).wait()
            pltpu.async_copy(
                bufs.at[1 - b],
                out_hbm.at[pl.ds(wbase + (cc - 1) * _K, _K)],
                sem_o[1 - b])

        pltpu.async_copy(tok_hbm.at[tok_idx_ref(u, slot)], bufs.at[b],
                         sem_t[b], add=True)

    def group(g, slot):
        wait_stage(g, slot)
        for u in range(_GC):
            cc = g * _GC + u
            step(cc, u % 2, u, slot)
            if u == 0:
                @pl.when(g + 1 < ngroup)
                def _():
                    fire_stage(g + 1, 1 - slot)

    fire_stage(0, 0)

    def gpair(gp, carry):
        group(gp * 2, 0)
        group(gp * 2 + 1, 1)
        return carry

    lax.fori_loop(0, ngroup // 2, gpair, 0)

    last = nchunk - 1
    pltpu.make_async_copy(tok_hbm.at[tok_idx_ref(_GC - 1, 1)],
                          bufs.at[1], sem_t[1]).wait()
    pltpu.async_copy(bufs.at[1],
                     out_hbm.at[pl.ds(wbase + last * _K, _K)], sem_o[1])
    pltpu.make_async_copy(
        bufs.at[0], out_hbm.at[pl.ds(wbase + (last - 1) * _K, _K)],
        sem_o[0]).wait()
    pltpu.make_async_copy(
        bufs.at[1], out_hbm.at[pl.ds(wbase + last * _K, _K)],
        sem_o[1]).wait()


def kernel(seq, seg, token_table, position_table, segment_table):
    batch, sent = seq.shape
    n = batch * sent
    seq_flat = seq.reshape(n).astype(jnp.int32)
    segf = seg.astype(jnp.float32).reshape(n)
    possum0 = position_table + segment_table[0][None, :]
    d = segment_table[1] - segment_table[0]

    mesh = plsc.VectorSubcoreMesh(core_axis_name="c", subcore_axis_name="s")
    run = functools.partial(
        pl.kernel,
        out_type=jax.ShapeDtypeStruct((n, _HIDDEN), jnp.float32),
        mesh=mesh,
        compiler_params=pltpu.CompilerParams(needs_layout_passes=False),
        scratch_types=[
            pltpu.VMEM((sent, _HIDDEN), jnp.float32),
            pltpu.VMEM((_HIDDEN,), jnp.float32),
            pltpu.VMEM((2, _GK), jnp.int32),
            pltpu.VMEM((_GK,), jnp.float32),
            pltpu.VMEM((_GK,), jnp.float32),
            pltpu.VMEM((_K, _LANES), jnp.float32),
            pltpu.VMEM((2, _K, _HIDDEN), jnp.float32),
        ] + [pltpu.SemaphoreType.DMA] * 8,
    )(_emb_body)
    out = run(seq_flat, segf, token_table, possum0, d)
    return out.reshape(batch, sent, _HIDDEN)


# speedup vs baseline: 1.6831x; 1.6831x over previous
import functools

import jax
import jax.numpy as jnp
from jax import lax
from jax.experimental import pallas as pl
from jax.experimental.pallas import tpu as pltpu
from jax.experimental.pallas import tpu_sc as plsc

_HIDDEN = 128
_LANES = 16
_NW = 32
_K = 128
_GC = 8
_GK = _GC * _K


def _emb_body(seq_hbm, segf_hbm, tok_hbm, possum_hbm, d_hbm, out_hbm,
              possum_v, d_v, idx_grp, segf_g0, segf_g1, bufs,
              sem_t0, sem_t1, sem_o0, sem_o1,
              sem_si0, sem_si1, sem_ss0, sem_ss1):
    sem_t = [sem_t0, sem_t1]
    sem_o = [sem_o0, sem_o1]
    sem_si = [sem_si0, sem_si1]
    sem_ss = [sem_ss0, sem_ss1]
    segf_grp = [segf_g0, segf_g1]
    n = out_hbm.shape[0]
    rows_per_w = n // _NW
    nchunk = rows_per_w // _K
    ngroup = nchunk // _GC
    sent = possum_v.shape[0]
    wid = lax.axis_index("s") * 2 + lax.axis_index("c")
    wbase = wid * rows_per_w

    pltpu.sync_copy(possum_hbm, possum_v)
    pltpu.sync_copy(d_hbm, d_v)
    dvs = [d_v[pl.ds(j * _LANES, _LANES)] for j in range(_HIDDEN // _LANES)]

    def fire_stage(g, slot):
        pltpu.async_copy(seq_hbm.at[pl.ds(wbase + g * _GK, _GK)],
                         idx_grp.at[slot], sem_si[slot])
        pltpu.async_copy(segf_hbm.at[pl.ds(wbase + g * _GK, _GK)],
                         segf_grp[slot], sem_ss[slot])

    def wait_stage(g, slot):
        pltpu.make_async_copy(seq_hbm.at[pl.ds(wbase + g * _GK, _GK)],
                              idx_grp.at[slot], sem_si[slot]).wait()
        pltpu.make_async_copy(segf_hbm.at[pl.ds(wbase + g * _GK, _GK)],
                              segf_grp[slot], sem_ss[slot]).wait()

    def prefill(b, u, slot):
        s_base = (u % (sent // _K)) * _K
        segbase = u * _K
        bv = bufs.at[b]
        sv = segf_grp[slot]

        @plsc.parallel_loop(0, _K, step=1, unroll=4)
        def _row(r):
            iv = jnp.full((_LANES,), segbase, jnp.int32) + lax.broadcast(
                r, (_LANES,))
            spl = plsc.load_gather(sv, [iv])
            for j in range(_HIDDEN // _LANES):
                sl = pl.ds(j * _LANES, _LANES)
                bv[r, sl] = possum_v[s_base + r, sl] + spl * dvs[j]

    def tok_idx_ref(u, slot):
        return idx_grp.at[slot, pl.ds(u * _K, _K)]

    def step(cc, b, u, slot):
        @pl.when(cc >= 2)
        def _():
            pltpu.make_async_copy(
                bufs.at[b],
                out_hbm.at[pl.ds(wbase + (cc - 2) * _K, _K)],
                sem_o[b]).wait()

        prefill(b, u, slot)

        if u > 0:
            iref = tok_idx_ref(u - 1, slot)
        else:
            iref = tok_idx_ref(_GC - 1, 1 - slot)

        @pl.when(cc >= 1)
        def _():
            pltpu.make_async_copy(tok_hbm.at[iref], bufs.at[1 - b],
                                  sem_t[1 - b]).wait()
            pltpu.async_copy(
                bufs.at[1 - b],
                out_hbm.at[pl.ds(wbase + (cc - 1) * _K, _K)],
                sem_o[1 - b])

        pltpu.async_copy(tok_hbm.at[tok_idx_ref(u, slot)], bufs.at[b],
                         sem_t[b], add=True)

    def group(g, slot):
        wait_stage(g, slot)
        for u in range(_GC):
            cc = g * _GC + u
            step(cc, u % 2, u, slot)
            if u == 0:
                @pl.when(g + 1 < ngroup)
                def _():
                    fire_stage(g + 1, 1 - slot)

    fire_stage(0, 0)

    def gpair(gp, carry):
        group(gp * 2, 0)
        group(gp * 2 + 1, 1)
        return carry

    lax.fori_loop(0, ngroup // 2, gpair, 0)

    last = nchunk - 1
    pltpu.make_async_copy(tok_hbm.at[tok_idx_ref(_GC - 1, 1)],
                          bufs.at[1], sem_t[1]).wait()
    pltpu.async_copy(bufs.at[1],
                     out_hbm.at[pl.ds(wbase + last * _K, _K)], sem_o[1])
    pltpu.make_async_copy(
        bufs.at[0], out_hbm.at[pl.ds(wbase + (last - 1) * _K, _K)],
        sem_o[0]).wait()
    pltpu.make_async_copy(
        bufs.at[1], out_hbm.at[pl.ds(wbase + last * _K, _K)],
        sem_o[1]).wait()


def kernel(seq, seg, token_table, position_table, segment_table):
    batch, sent = seq.shape
    n = batch * sent
    seq_flat = seq.reshape(n).astype(jnp.int32)
    segf = seg.astype(jnp.float32).reshape(n)
    possum0 = position_table + segment_table[0][None, :]
    d = segment_table[1] - segment_table[0]

    mesh = plsc.VectorSubcoreMesh(core_axis_name="c", subcore_axis_name="s")
    run = functools.partial(
        pl.kernel,
        out_type=jax.ShapeDtypeStruct((n, _HIDDEN), jnp.float32),
        mesh=mesh,
        compiler_params=pltpu.CompilerParams(needs_layout_passes=False),
        scratch_types=[
            pltpu.VMEM((sent, _HIDDEN), jnp.float32),
            pltpu.VMEM((_HIDDEN,), jnp.float32),
            pltpu.VMEM((2, _GK), jnp.int32),
            pltpu.VMEM((_GK,), jnp.float32),
            pltpu.VMEM((_GK,), jnp.float32),
            pltpu.VMEM((2, _K, _HIDDEN), jnp.float32),
        ] + [pltpu.SemaphoreType.DMA] * 8,
    )(_emb_body)
    out = run(seq_flat, segf, token_table, possum0, d)
    return out.reshape(batch, sent, _HIDDEN)


# speedup vs baseline: 1.8802x; 1.1172x over previous
import functools

import jax
import jax.numpy as jnp
from jax import lax
from jax.experimental import pallas as pl
from jax.experimental.pallas import tpu as pltpu
from jax.experimental.pallas import tpu_sc as plsc

_HIDDEN = 128
_LANES = 16
_NW = 32
_K = 128
_GC = 8
_GK = _GC * _K


def _emb_body(seq_hbm, segf_hbm, tok_hbm, possum_hbm, d_hbm, out_hbm,
              possum_v, d_v, idx_grp, segf_g0, segf_g1, bufs,
              sem_t0, sem_t1, sem_o0, sem_o1,
              sem_si0, sem_si1, sem_ss0, sem_ss1):
    sem_t = [sem_t0, sem_t1]
    sem_o = [sem_o0, sem_o1]
    sem_si = [sem_si0, sem_si1]
    sem_ss = [sem_ss0, sem_ss1]
    segf_grp = [segf_g0, segf_g1]
    n = out_hbm.shape[0]
    rows_per_w = n // _NW
    nchunk = rows_per_w // _K
    ngroup = nchunk // _GC
    sent = possum_v.shape[0]
    wid = lax.axis_index("s") * 2 + lax.axis_index("c")
    wbase = wid * rows_per_w

    pltpu.sync_copy(possum_hbm, possum_v)
    pltpu.sync_copy(d_hbm, d_v)
    dvs = [d_v[pl.ds(j * _LANES, _LANES)] for j in range(_HIDDEN // _LANES)]

    def fire_stage(g, slot):
        pltpu.async_copy(seq_hbm.at[pl.ds(wbase + g * _GK, _GK)],
                         idx_grp.at[slot], sem_si[slot])
        pltpu.async_copy(segf_hbm.at[pl.ds(wbase + g * _GK, _GK)],
                         segf_grp[slot], sem_ss[slot])

    def wait_stage(g, slot):
        pltpu.make_async_copy(seq_hbm.at[pl.ds(wbase + g * _GK, _GK)],
                              idx_grp.at[slot], sem_si[slot]).wait()
        pltpu.make_async_copy(segf_hbm.at[pl.ds(wbase + g * _GK, _GK)],
                              segf_grp[slot], sem_ss[slot]).wait()

    def prefill(b, u, slot):
        s_base = (u % (sent // _K)) * _K
        segbase = u * _K
        bv = bufs.at[b]
        sv = segf_grp[slot]

        @plsc.parallel_loop(0, _K, step=1, unroll=4)
        def _row(r):
            iv = jnp.full((_LANES,), segbase, jnp.int32) + lax.broadcast(
                r, (_LANES,))
            spl = plsc.load_gather(sv, [iv])
            for j in range(_HIDDEN // _LANES):
                sl = pl.ds(j * _LANES, _LANES)
                bv[r, sl] = possum_v[s_base + r, sl] + spl * dvs[j]

    def tok_idx_ref(u, slot):
        return idx_grp.at[slot, pl.ds(u * _K, _K)]

    def step(cc, b, u, slot):
        @pl.when(cc >= 2)
        def _():
            pltpu.make_async_copy(
                bufs.at[b],
                out_hbm.at[pl.ds(wbase + (cc - 2) * _K, _K)],
                sem_o[b]).wait()

        prefill(b, u, slot)
        pltpu.async_copy(tok_hbm.at[tok_idx_ref(u, slot)], bufs.at[b],
                         sem_t[b], add=True)

        if u > 0:
            iref = tok_idx_ref(u - 1, slot)
        else:
            iref = tok_idx_ref(_GC - 1, 1 - slot)

        @pl.when(cc >= 1)
        def _():
            pltpu.make_async_copy(tok_hbm.at[iref], bufs.at[1 - b],
                                  sem_t[1 - b]).wait()
            pltpu.async_copy(
                bufs.at[1 - b],
                out_hbm.at[pl.ds(wbase + (cc - 1) * _K, _K)],
                sem_o[1 - b])

    def group(g, slot):
        wait_stage(g, slot)
        for u in range(_GC):
            cc = g * _GC + u
            step(cc, u % 2, u, slot)
            if u == 0:
                @pl.when(g + 1 < ngroup)
                def _():
                    fire_stage(g + 1, 1 - slot)

    fire_stage(0, 0)

    def gpair(gp, carry):
        group(gp * 2, 0)
        group(gp * 2 + 1, 1)
        return carry

    lax.fori_loop(0, ngroup // 2, gpair, 0)

    last = nchunk - 1
    pltpu.make_async_copy(tok_hbm.at[tok_idx_ref(_GC - 1, 1)],
                          bufs.at[1], sem_t[1]).wait()
    pltpu.async_copy(bufs.at[1],
                     out_hbm.at[pl.ds(wbase + last * _K, _K)], sem_o[1])
    pltpu.make_async_copy(
        bufs.at[0], out_hbm.at[pl.ds(wbase + (last - 1) * _K, _K)],
        sem_o[0]).wait()
    pltpu.make_async_copy(
        bufs.at[1], out_hbm.at[pl.ds(wbase + last * _K, _K)],
        sem_o[1]).wait()


def kernel(seq, seg, token_table, position_table, segment_table):
    batch, sent = seq.shape
    n = batch * sent
    seq_flat = seq.reshape(n).astype(jnp.int32)
    segf = seg.astype(jnp.float32).reshape(n)
    possum0 = position_table + segment_table[0][None, :]
    d = segment_table[1] - segment_table[0]

    mesh = plsc.VectorSubcoreMesh(core_axis_name="c", subcore_axis_name="s")
    run = functools.partial(
        pl.kernel,
        out_type=jax.ShapeDtypeStruct((n, _HIDDEN), jnp.float32),
        mesh=mesh,
        compiler_params=pltpu.CompilerParams(needs_layout_passes=False),
        scratch_types=[
            pltpu.VMEM((sent, _HIDDEN), jnp.float32),
            pltpu.VMEM((_HIDDEN,), jnp.float32),
            pltpu.VMEM((2, _GK), jnp.int32),
            pltpu.VMEM((_GK,), jnp.float32),
            pltpu.VMEM((_GK,), jnp.float32),
            pltpu.VMEM((2, _K, _HIDDEN), jnp.float32),
        ] + [pltpu.SemaphoreType.DMA] * 8,
    )(_emb_body)
    out = run(seq_flat, segf, token_table, possum0, d)
    return out.reshape(batch, sent, _HIDDEN)


# speedup vs baseline: 2.0206x; 1.0747x over previous
import functools

import jax
import jax.numpy as jnp
from jax import lax
from jax.experimental import pallas as pl
from jax.experimental.pallas import tpu as pltpu
from jax.experimental.pallas import tpu_sc as plsc

_HIDDEN = 128
_LANES = 16
_NW = 32
_K = 64
_NBUF = 4
_GC = 8
_GK = _GC * _K


def _emb_body(seq_hbm, segf_hbm, tok_hbm, possum_hbm, d_hbm, out_hbm,
              possum_v, d_v, idx_grp, segf_g0, segf_g1, bufs,
              sem_t0, sem_t1, sem_t2, sem_t3,
              sem_o0, sem_o1, sem_o2, sem_o3,
              sem_si0, sem_si1, sem_ss0, sem_ss1):
    sem_t = [sem_t0, sem_t1, sem_t2, sem_t3]
    sem_o = [sem_o0, sem_o1, sem_o2, sem_o3]
    sem_si = [sem_si0, sem_si1]
    sem_ss = [sem_ss0, sem_ss1]
    segf_grp = [segf_g0, segf_g1]
    n = out_hbm.shape[0]
    rows_per_w = n // _NW
    nchunk = rows_per_w // _K
    ngroup = nchunk // _GC
    wid = lax.axis_index("s") * 2 + lax.axis_index("c")
    wbase = wid * rows_per_w

    pltpu.sync_copy(possum_hbm, possum_v)
    pltpu.sync_copy(d_hbm, d_v)
    dvs = [d_v[pl.ds(j * _LANES, _LANES)] for j in range(_HIDDEN // _LANES)]

    def fire_stage(g, slot):
        pltpu.async_copy(seq_hbm.at[pl.ds(wbase + g * _GK, _GK)],
                         idx_grp.at[slot], sem_si[slot])
        pltpu.async_copy(segf_hbm.at[pl.ds(wbase + g * _GK, _GK)],
                         segf_grp[slot], sem_ss[slot])

    def wait_stage(g, slot):
        pltpu.make_async_copy(seq_hbm.at[pl.ds(wbase + g * _GK, _GK)],
                              idx_grp.at[slot], sem_si[slot]).wait()
        pltpu.make_async_copy(segf_hbm.at[pl.ds(wbase + g * _GK, _GK)],
                              segf_grp[slot], sem_ss[slot]).wait()

    def prefill(b, u, slot):
        s_base = u * _K
        segbase = u * _K
        bv = bufs.at[b]
        sv = segf_grp[slot]

        @plsc.parallel_loop(0, _K, step=1, unroll=4)
        def _row(r):
            iv = jnp.full((_LANES,), segbase, jnp.int32) + lax.broadcast(
                r, (_LANES,))
            spl = plsc.load_gather(sv, [iv])
            for j in range(_HIDDEN // _LANES):
                sl = pl.ds(j * _LANES, _LANES)
                bv[r, sl] = possum_v[s_base + r, sl] + spl * dvs[j]

    def tok_idx_ref(u, slot):
        return idx_grp.at[slot, pl.ds(u * _K, _K)]

    def step(cc, b, u, slot):
        @pl.when(cc >= _NBUF)
        def _():
            pltpu.make_async_copy(
                bufs.at[b],
                out_hbm.at[pl.ds(wbase + (cc - _NBUF) * _K, _K)],
                sem_o[b]).wait()

        prefill(b, u, slot)
        pltpu.async_copy(tok_hbm.at[tok_idx_ref(u, slot)], bufs.at[b],
                         sem_t[b], add=True)

        if u >= 2:
            iref = tok_idx_ref(u - 2, slot)
        else:
            iref = tok_idx_ref(u + _GC - 2, 1 - slot)
        b2 = (b - 2) % _NBUF

        @pl.when(cc >= 2)
        def _():
            pltpu.make_async_copy(tok_hbm.at[iref], bufs.at[b2],
                                  sem_t[b2]).wait()
            pltpu.async_copy(
                bufs.at[b2],
                out_hbm.at[pl.ds(wbase + (cc - 2) * _K, _K)],
                sem_o[b2])

    def group(g, slot):
        for u in range(_GC):
            cc = g * _GC + u
            if u == _GC - 1:
                @pl.when(g + 1 < ngroup)
                def _():
                    wait_stage(g + 1, 1 - slot)

            step(cc, u % _NBUF, u, slot)
            if u == 1:
                @pl.when(g + 1 < ngroup)
                def _():
                    fire_stage(g + 1, 1 - slot)

    fire_stage(0, 0)
    wait_stage(0, 0)

    def gpair(gp, carry):
        group(gp * 2, 0)
        group(gp * 2 + 1, 1)
        return carry

    lax.fori_loop(0, ngroup // 2, gpair, 0)

    for k in (2, 1):
        cc = nchunk - k
        u = _GC - k
        b = u % _NBUF
        pltpu.make_async_copy(tok_hbm.at[tok_idx_ref(u, 1)],
                              bufs.at[b], sem_t[b]).wait()
        pltpu.async_copy(
            bufs.at[b], out_hbm.at[pl.ds(wbase + cc * _K, _K)], sem_o[b])
    for k in (4, 3, 2, 1):
        cc = nchunk - k
        b = (_GC - k) % _NBUF
        pltpu.make_async_copy(
            bufs.at[b], out_hbm.at[pl.ds(wbase + cc * _K, _K)],
            sem_o[b]).wait()


def kernel(seq, seg, token_table, position_table, segment_table):
    batch, sent = seq.shape
    n = batch * sent
    seq_flat = seq.reshape(n).astype(jnp.int32)
    segf = seg.astype(jnp.float32).reshape(n)
    possum0 = position_table + segment_table[0][None, :]
    d = segment_table[1] - segment_table[0]

    mesh = plsc.VectorSubcoreMesh(core_axis_name="c", subcore_axis_name="s")
    run = functools.partial(
        pl.kernel,
        out_type=jax.ShapeDtypeStruct((n, _HIDDEN), jnp.float32),
        mesh=mesh,
        compiler_params=pltpu.CompilerParams(needs_layout_passes=False),
        scratch_types=[
            pltpu.VMEM((sent, _HIDDEN), jnp.float32),
            pltpu.VMEM((_HIDDEN,), jnp.float32),
            pltpu.VMEM((2, _GK), jnp.int32),
            pltpu.VMEM((_GK,), jnp.float32),
            pltpu.VMEM((_GK,), jnp.float32),
            pltpu.VMEM((_NBUF, _K, _HIDDEN), jnp.float32),
        ] + [pltpu.SemaphoreType.DMA] * 12,
    )(_emb_body)
    out = run(seq_flat, segf, token_table, possum0, d)
    return out.reshape(batch, sent, _HIDDEN)
